# baseline (speedup 1.0000x reference)
import functools

import jax
import jax.numpy as jnp
from jax import lax
from jax.experimental import pallas as pl
from jax.experimental.pallas import tpu as pltpu
from jax.experimental.pallas import tpu_sc as plsc

B = 16384
NUM_DIM = 13
N_CAT = 26
VOCAB = 100000
EMB = 16
OUT = 128
TOTAL = NUM_DIM + N_CAT * EMB
HID = TOTAL * 2

NJ = N_CAT * EMB
_NW = 32
_JPW = NJ // _NW
_HALF = B // 2
_L = 16
_UNROLL = 8

_BM = 2048


def _gather_t(tabT, catT):
    mesh = plsc.VectorSubcoreMesh(core_axis_name="core", subcore_axis_name="subcore")

    @functools.partial(
        pl.kernel,
        out_type=jax.ShapeDtypeStruct((NJ, B), jnp.float32),
        mesh=mesh,
        compiler_params=pltpu.CompilerParams(
            use_tc_tiling_on_sc=True, needs_layout_passes=False
        ),
        scratch_types=[
            pltpu.VMEM((VOCAB,), jnp.float32),
            pltpu.VMEM((B,), jnp.int32),
            pltpu.VMEM((_HALF,), jnp.float32),
            pltpu.SemaphoreType.DMA,
            pltpu.SemaphoreType.DMA,
        ],
    )
    def k(t_hbm, i_hbm, o_hbm, row_v, idx_v, outh_v, lsem, wsem):
        wid = lax.axis_index("subcore") * 2 + lax.axis_index("core")

        @pl.loop(0, _JPW)
        def _(jj):
            j = wid * _JPW + jj
            f = j // EMB
            a_idx = pltpu.async_copy(i_hbm.at[f], idx_v, lsem)
            a_row = pltpu.async_copy(t_hbm.at[j], row_v, lsem)
            a_idx.wait()
            a_row.wait()

            for h in range(2):
                if h == 1:
                    pltpu.make_async_copy(
                        outh_v, o_hbm.at[0].at[pl.ds(0, _HALF)], wsem
                    ).wait()
                else:

                    @pl.when(jj > 0)
                    def _():
                        pltpu.make_async_copy(
                            outh_v, o_hbm.at[0].at[pl.ds(0, _HALF)], wsem
                        ).wait()

                @pl.loop(0, _HALF // (_L * _UNROLL))
                def _(kk):
                    base = kk * (_L * _UNROLL)
                    for u in range(_UNROLL):
                        o = base + u * _L
                        idx16 = idx_v[pl.ds(h * _HALF + o, _L)]
                        outh_v[pl.ds(o, _L)] = plsc.load_gather(row_v, [idx16])

                pltpu.async_copy(
                    outh_v, o_hbm.at[j].at[pl.ds(h * _HALF, _HALF)], wsem
                )

        pltpu.make_async_copy(
            outh_v, o_hbm.at[0].at[pl.ds(0, _HALF)], wsem
        ).wait()

    return k(tabT, catT)


def _mlp_body(numT_ref, catT_ref, w1n_ref, w1c_ref, b1_ref, w2_ref, b2_ref, out_ref):
    bf = jnp.bfloat16
    cdims = (((0,), (0,)), ((), ()))
    h = lax.dot_general(
        catT_ref[...].astype(bf),
        w1c_ref[...].astype(bf),
        cdims,
        preferred_element_type=jnp.float32,
    )
    h += lax.dot_general(
        numT_ref[...].astype(bf),
        w1n_ref[...].astype(bf),
        cdims,
        preferred_element_type=jnp.float32,
    )
    h = jnp.maximum(h + b1_ref[...], 0.0)
    out_ref[...] = (
        jnp.dot(
            h.astype(bf), w2_ref[...].astype(bf), preferred_element_type=jnp.float32
        )
        + b2_ref[...]
    )


def _mlp(numT, catT, w1n, w1c, b1, w2, b2):
    grid = (B // _BM,)
    return pl.pallas_call(
        _mlp_body,
        grid=grid,
        in_specs=[
            pl.BlockSpec((NUM_DIM, _BM), lambda i: (0, i)),
            pl.BlockSpec((NJ, _BM), lambda i: (0, i)),
            pl.BlockSpec((NUM_DIM, HID), lambda i: (0, 0)),
            pl.BlockSpec((NJ, HID), lambda i: (0, 0)),
            pl.BlockSpec((1, HID), lambda i: (0, 0)),
            pl.BlockSpec((HID, OUT), lambda i: (0, 0)),
            pl.BlockSpec((1, OUT), lambda i: (0, 0)),
        ],
        out_specs=pl.BlockSpec((_BM, OUT), lambda i: (i, 0)),
        out_shape=jax.ShapeDtypeStruct((B, OUT), jnp.float32),
    )(numT, catT, w1n, w1c, b1, w2, b2)


def kernel(numerical_feats, categorical_feats, emb, W1, b1, W2, b2):
    tabT = emb.transpose(0, 2, 1).reshape(NJ, VOCAB)
    catT = categorical_feats.T
    outT = _gather_t(tabT, catT)
    numT = numerical_feats.T
    w1n = W1[:NUM_DIM]
    w1c = W1[NUM_DIM:]
    return _mlp(
        numT,
        outT,
        w1n,
        w1c,
        b1.reshape(1, HID),
        W2,
        b2.reshape(1, OUT),
    )


# speedup vs baseline: 1.0022x; 1.0022x over previous
import functools

import jax
import jax.numpy as jnp
from jax import lax
from jax.experimental import pallas as pl
from jax.experimental.pallas import tpu as pltpu
from jax.experimental.pallas import tpu_sc as plsc

B = 16384
NUM_DIM = 13
N_CAT = 26
VOCAB = 100000
EMB = 16
OUT = 128
TOTAL = NUM_DIM + N_CAT * EMB
HID = TOTAL * 2

NJ = N_CAT * EMB
_NW = 32
_JPW = NJ // _NW
_HALF = B // 2
_L = 16
_UNROLL = 8

_BM = 2048


def _gather_t(tabT, catT):
    mesh = plsc.VectorSubcoreMesh(core_axis_name="core", subcore_axis_name="subcore")

    @functools.partial(
        pl.kernel,
        out_type=jax.ShapeDtypeStruct((NJ, B), jnp.float32),
        mesh=mesh,
        compiler_params=pltpu.CompilerParams(
            use_tc_tiling_on_sc=True, needs_layout_passes=False
        ),
        scratch_types=[
            pltpu.VMEM((VOCAB,), jnp.float32),
            pltpu.VMEM((B,), jnp.int32),
            pltpu.VMEM((_HALF,), jnp.float32),
            pltpu.SemaphoreType.DMA,
            pltpu.SemaphoreType.DMA,
        ],
    )
    def k(t_hbm, i_hbm, o_hbm, row_v, idx_v, outh_v, lsem, wsem):
        wid = lax.axis_index("subcore") * 2 + lax.axis_index("core")

        @pl.loop(0, _JPW)
        def _(jj):
            j = wid * _JPW + jj
            f = j // EMB
            a_idx = pltpu.async_copy(i_hbm.at[f], idx_v, lsem)
            a_row = pltpu.async_copy(t_hbm.at[j], row_v, lsem)
            a_idx.wait()
            a_row.wait()

            for h in range(2):
                if h == 1:
                    pltpu.make_async_copy(
                        outh_v, o_hbm.at[0].at[pl.ds(0, _HALF)], wsem
                    ).wait()
                else:

                    @pl.when(jj > 0)
                    def _():
                        pltpu.make_async_copy(
                            outh_v, o_hbm.at[0].at[pl.ds(0, _HALF)], wsem
                        ).wait()

                @pl.loop(0, _HALF // (_L * _UNROLL))
                def _(kk):
                    base = kk * (_L * _UNROLL)
                    for u in range(_UNROLL):
                        o = base + u * _L
                        idx16 = idx_v[pl.ds(h * _HALF + o, _L)]
                        outh_v[pl.ds(o, _L)] = plsc.load_gather(row_v, [idx16])

                pltpu.async_copy(
                    outh_v, o_hbm.at[j].at[pl.ds(h * _HALF, _HALF)], wsem
                )

        pltpu.make_async_copy(
            outh_v, o_hbm.at[0].at[pl.ds(0, _HALF)], wsem
        ).wait()

    return k(tabT, catT)


def _mlp_body(numT_ref, catT_ref, w1n_ref, w1c_ref, b1_ref, w2_ref, b2_ref, out_ref):
    cdims = (((0,), (0,)), ((), ()))
    h = lax.dot_general(
        catT_ref[...], w1c_ref[...], cdims, preferred_element_type=jnp.float32
    )
    h += lax.dot_general(
        numT_ref[...], w1n_ref[...], cdims, preferred_element_type=jnp.float32
    )
    h = jnp.maximum(h + b1_ref[...], 0.0)
    out_ref[...] = (
        jnp.dot(h, w2_ref[...], preferred_element_type=jnp.float32) + b2_ref[...]
    )


def _mlp(numT, catT, w1n, w1c, b1, w2, b2):
    grid = (B // _BM,)
    return pl.pallas_call(
        _mlp_body,
        grid=grid,
        in_specs=[
            pl.BlockSpec((NUM_DIM, _BM), lambda i: (0, i)),
            pl.BlockSpec((NJ, _BM), lambda i: (0, i)),
            pl.BlockSpec((NUM_DIM, HID), lambda i: (0, 0)),
            pl.BlockSpec((NJ, HID), lambda i: (0, 0)),
            pl.BlockSpec((1, HID), lambda i: (0, 0)),
            pl.BlockSpec((HID, OUT), lambda i: (0, 0)),
            pl.BlockSpec((1, OUT), lambda i: (0, 0)),
        ],
        out_specs=pl.BlockSpec((_BM, OUT), lambda i: (i, 0)),
        out_shape=jax.ShapeDtypeStruct((B, OUT), jnp.float32),
    )(numT, catT, w1n, w1c, b1, w2, b2)


def kernel(numerical_feats, categorical_feats, emb, W1, b1, W2, b2):
    tabT = emb.transpose(0, 2, 1).reshape(NJ, VOCAB)
    catT = categorical_feats.T
    outT = _gather_t(tabT, catT)
    numT = numerical_feats.T
    w1n = W1[:NUM_DIM]
    w1c = W1[NUM_DIM:]
    return _mlp(
        numT,
        outT,
        w1n,
        w1c,
        b1.reshape(1, HID),
        W2,
        b2.reshape(1, OUT),
    )


# speedup vs baseline: 1.0064x; 1.0041x over previous
import functools

import jax
import jax.numpy as jnp
from jax import lax
from jax.experimental import pallas as pl
from jax.experimental.pallas import tpu as pltpu
from jax.experimental.pallas import tpu_sc as plsc

B = 16384
NUM_DIM = 13
N_CAT = 26
VOCAB = 100000
EMB = 16
OUT = 128
TOTAL = NUM_DIM + N_CAT * EMB
HID = TOTAL * 2

NJ = N_CAT * EMB
_NW = 32
_JPW = NJ // _NW
_HALF = B // 2
_L = 16
_UNROLL = 16

_BM = 2048


def _gather_t(tabT, catT):
    mesh = plsc.VectorSubcoreMesh(core_axis_name="core", subcore_axis_name="subcore")

    @functools.partial(
        pl.kernel,
        out_type=jax.ShapeDtypeStruct((NJ, B), jnp.float32),
        mesh=mesh,
        compiler_params=pltpu.CompilerParams(
            use_tc_tiling_on_sc=True, needs_layout_passes=False
        ),
        scratch_types=[
            pltpu.VMEM((VOCAB,), jnp.float32),
            pltpu.VMEM((B,), jnp.int32),
            pltpu.VMEM((_HALF,), jnp.float32),
            pltpu.SemaphoreType.DMA,
            pltpu.SemaphoreType.DMA,
        ],
    )
    def k(t_hbm, i_hbm, o_hbm, row_v, idx_v, outh_v, lsem, wsem):
        wid = lax.axis_index("subcore") * 2 + lax.axis_index("core")

        @pl.loop(0, _JPW)
        def _(jj):
            j = wid * _JPW + jj
            f = j // EMB
            a_idx = pltpu.async_copy(i_hbm.at[f], idx_v, lsem)
            a_row = pltpu.async_copy(t_hbm.at[j], row_v, lsem)
            a_idx.wait()
            a_row.wait()

            for h in range(2):
                if h == 1:
                    pltpu.make_async_copy(
                        outh_v, o_hbm.at[0].at[pl.ds(0, _HALF)], wsem
                    ).wait()
                else:

                    @pl.when(jj > 0)
                    def _():
                        pltpu.make_async_copy(
                            outh_v, o_hbm.at[0].at[pl.ds(0, _HALF)], wsem
                        ).wait()

                @pl.loop(0, _HALF // (_L * _UNROLL))
                def _(kk):
                    base = kk * (_L * _UNROLL)
                    for u in range(_UNROLL):
                        o = base + u * _L
                        idx16 = idx_v[pl.ds(h * _HALF + o, _L)]
                        outh_v[pl.ds(o, _L)] = plsc.load_gather(row_v, [idx16])

                pltpu.async_copy(
                    outh_v, o_hbm.at[j].at[pl.ds(h * _HALF, _HALF)], wsem
                )

        pltpu.make_async_copy(
            outh_v, o_hbm.at[0].at[pl.ds(0, _HALF)], wsem
        ).wait()

    return k(tabT, catT)


def _mlp_body(numT_ref, catT_ref, w1n_ref, w1c_ref, b1_ref, w2_ref, b2_ref, out_ref):
    cdims = (((0,), (0,)), ((), ()))
    h = lax.dot_general(
        catT_ref[...], w1c_ref[...], cdims, preferred_element_type=jnp.float32
    )
    h += lax.dot_general(
        numT_ref[...], w1n_ref[...], cdims, preferred_element_type=jnp.float32
    )
    h = jnp.maximum(h + b1_ref[...], 0.0)
    out_ref[...] = (
        jnp.dot(h, w2_ref[...], preferred_element_type=jnp.float32) + b2_ref[...]
    )


def _mlp(numT, catT, w1n, w1c, b1, w2, b2):
    grid = (B // _BM,)
    return pl.pallas_call(
        _mlp_body,
        grid=grid,
        in_specs=[
            pl.BlockSpec((NUM_DIM, _BM), lambda i: (0, i)),
            pl.BlockSpec((NJ, _BM), lambda i: (0, i)),
            pl.BlockSpec((NUM_DIM, HID), lambda i: (0, 0)),
            pl.BlockSpec((NJ, HID), lambda i: (0, 0)),
            pl.BlockSpec((1, HID), lambda i: (0, 0)),
            pl.BlockSpec((HID, OUT), lambda i: (0, 0)),
            pl.BlockSpec((1, OUT), lambda i: (0, 0)),
        ],
        out_specs=pl.BlockSpec((_BM, OUT), lambda i: (i, 0)),
        out_shape=jax.ShapeDtypeStruct((B, OUT), jnp.float32),
    )(numT, catT, w1n, w1c, b1, w2, b2)


def kernel(numerical_feats, categorical_feats, emb, W1, b1, W2, b2):
    tabT = emb.transpose(0, 2, 1).reshape(NJ, VOCAB)
    catT = categorical_feats.T
    outT = _gather_t(tabT, catT)
    numT = numerical_feats.T
    w1n = W1[:NUM_DIM]
    w1c = W1[NUM_DIM:]
    return _mlp(
        numT,
        outT,
        w1n,
        w1c,
        b1.reshape(1, HID),
        W2,
        b2.reshape(1, OUT),
    )
